# baseline (speedup 1.0000x reference)
import functools

import jax
import jax.numpy as jnp
from jax import lax
from jax.experimental import pallas as pl
from jax.experimental.pallas import tpu as pltpu
from jax.experimental.pallas import tpu_sc as plsc


def _argmin_body(k_codes, x_ref, w_ref, idx_ref):
    xb = x_ref[0]
    w = w_ref[...]
    st = lax.dot_general(w, xb, (((0,), (0,)), ((), ())),
                         preferred_element_type=jnp.float32)
    x2 = jnp.sum(xb * xb, axis=0)
    e2 = jnp.sum(w * w, axis=0)
    dist = (x2[None, :] - 2.0 * st) + e2[:, None]
    m = jnp.min(dist, axis=0, keepdims=True)
    kiota = lax.broadcasted_iota(jnp.int32, dist.shape, 0)
    am = jnp.min(jnp.where(dist == m, kiota, k_codes), axis=0)
    idx_ref[0, 0, :] = am


def _argmin_call(x3, weight):
    b, d, p = x3.shape
    k = weight.shape[1]
    return pl.pallas_call(
        functools.partial(_argmin_body, k),
        grid=(b,),
        in_specs=[
            pl.BlockSpec((1, d, p), lambda i: (i, 0, 0)),
            pl.BlockSpec((d, k), lambda i: (0, 0)),
        ],
        out_specs=pl.BlockSpec((1, 1, p), lambda i: (i, 0, 0)),
        out_shape=jax.ShapeDtypeStruct((b, 1, p), jnp.int32),
    )(x3, weight)


def _sc_gather_dmajor(weight, idx, b, p):
    nc, ns = 2, 16
    nw = nc * ns
    dd, k = weight.shape
    wpb = nw // b
    drows = dd // wpb
    dc = 16
    n_dc = drows // dc
    npg = p // 16
    mesh = plsc.VectorSubcoreMesh(core_axis_name="c", subcore_axis_name="s",
                                  num_cores=nc, num_subcores=ns)

    @functools.partial(
        pl.kernel, mesh=mesh,
        compiler_params=pltpu.CompilerParams(needs_layout_passes=False),
        out_type=jax.ShapeDtypeStruct((b * dd, p), jnp.float32),
        scratch_types=[
            pltpu.VMEM((p,), jnp.int32),
            pltpu.VMEM((dc, k), jnp.float32),
            pltpu.VMEM((dc, k), jnp.float32),
            pltpu.VMEM((dc, p), jnp.float32),
            pltpu.VMEM((dc, p), jnp.float32),
            pltpu.SemaphoreType.DMA,
            pltpu.SemaphoreType.DMA,
            pltpu.SemaphoreType.DMA,
            pltpu.SemaphoreType.DMA,
        ],
    )
    def gather_kernel(w_hbm, idx_hbm, out_hbm, idx_v, wb0, wb1, ob0, ob1,
                      si0, si1, so0, so1):
        wid = lax.axis_index("s") * nc + lax.axis_index("c")
        bb = wid // wpb
        d0 = (wid % wpb) * drows
        wbufs = ((wb0, si0), (wb1, si1))
        obufs = ((ob0, so0), (ob1, so1))
        in_copies = [None] * n_dc
        out_copies = [None] * n_dc
        in_copies[0] = pltpu.async_copy(
            w_hbm.at[pl.ds(d0, dc)], wbufs[0][0], wbufs[0][1])
        pltpu.sync_copy(idx_hbm.at[pl.ds(bb * p, p)], idx_v)
        for c in range(n_dc):
            wb, _ = wbufs[c % 2]
            ob, so = obufs[c % 2]
            if c + 1 < n_dc:
                nwb, nsi = wbufs[(c + 1) % 2]
                in_copies[c + 1] = pltpu.async_copy(
                    w_hbm.at[pl.ds(d0 + (c + 1) * dc, dc)], nwb, nsi)
            in_copies[c].wait()
            if c >= 2:
                out_copies[c - 2].wait()

            def body(pg, _):
                iv = idx_v[pl.ds(pg * 16, 16)]
                for dl in range(dc):
                    row = jnp.full((16,), dl, jnp.int32)
                    ob[dl, pl.ds(pg * 16, 16)] = plsc.load_gather(
                        wb, [row, iv])
                return 0

            lax.fori_loop(0, npg, body, 0)
            out_copies[c] = pltpu.async_copy(
                ob, out_hbm.at[pl.ds(bb * dd + d0 + c * dc, dc)], so)
        for c in range(max(0, n_dc - 2), n_dc):
            out_copies[c].wait()

    return gather_kernel(weight, idx)


def kernel(x, weight):
    b, d, h, w = x.shape
    p = h * w
    x3 = x.reshape(b, d, p)
    idx3 = _argmin_call(x3, weight)
    res = _sc_gather_dmajor(weight, idx3.reshape(b * p), b, p)
    return res.reshape(b, d, h, w), idx3.reshape(b, h, w)


# speedup vs baseline: 1.1546x; 1.1546x over previous
import functools

import jax
import jax.numpy as jnp
from jax import lax
from jax.experimental import pallas as pl
from jax.experimental.pallas import tpu as pltpu
from jax.experimental.pallas import tpu_sc as plsc


def _argmin_body(k_codes, x_ref, w_ref, idx_ref, wt_ref):
    xb = x_ref[0]
    w = w_ref[...]
    st = lax.dot_general(w, xb, (((0,), (0,)), ((), ())),
                         preferred_element_type=jnp.float32)
    x2 = jnp.sum(xb * xb, axis=0)
    e2 = jnp.sum(w * w, axis=0)
    dist = (x2[None, :] - 2.0 * st) + e2[:, None]
    m = jnp.min(dist, axis=0, keepdims=True)
    kiota = lax.broadcasted_iota(jnp.int32, dist.shape, 0)
    am = jnp.min(jnp.where(dist == m, kiota, k_codes), axis=0)
    idx_ref[0, 0, :] = am

    @pl.when(pl.program_id(0) == 0)
    def _():
        wt_ref[...] = w.T


def _argmin_call(x3, weight):
    b, d, p = x3.shape
    k = weight.shape[1]
    return pl.pallas_call(
        functools.partial(_argmin_body, k),
        grid=(b,),
        in_specs=[
            pl.BlockSpec((1, d, p), lambda i: (i, 0, 0)),
            pl.BlockSpec((d, k), lambda i: (0, 0)),
        ],
        out_specs=[
            pl.BlockSpec((1, 1, p), lambda i: (i, 0, 0)),
            pl.BlockSpec((k, d), lambda i: (0, 0)),
        ],
        out_shape=[
            jax.ShapeDtypeStruct((b, 1, p), jnp.int32),
            jax.ShapeDtypeStruct((k, d), jnp.float32),
        ],
    )(x3, weight)


def _sc_gather(wt, idx):
    nc, ns = 2, 16
    nw = nc * ns
    n, d = idx.shape[0], wt.shape[1]
    b_per_w = n // nw
    ch = min(128, b_per_w)
    n_ch = b_per_w // ch
    mesh = plsc.VectorSubcoreMesh(core_axis_name="c", subcore_axis_name="s",
                                  num_cores=nc, num_subcores=ns)

    @functools.partial(
        pl.kernel, mesh=mesh,
        out_type=jax.ShapeDtypeStruct((n, d), jnp.float32),
        scratch_types=[
            pltpu.VMEM((b_per_w,), jnp.int32),
            pltpu.VMEM((ch, d), jnp.float32),
            pltpu.VMEM((ch, d), jnp.float32),
            pltpu.SemaphoreType.DMA,
            pltpu.SemaphoreType.DMA,
        ],
    )
    def gather_kernel(table_hbm, idx_hbm, out_hbm, idx_v, rows0, rows1,
                      sem0, sem1):
        wid = lax.axis_index("s") * nc + lax.axis_index("c")
        base = wid * b_per_w
        pltpu.sync_copy(idx_hbm.at[pl.ds(base, b_per_w)], idx_v)
        bufs = ((rows0, sem0), (rows1, sem1))
        copies = [None] * n_ch
        for c in range(n_ch):
            rows, sem = bufs[c % 2]
            copies[c] = pltpu.async_copy(
                table_hbm.at[idx_v.at[pl.ds(c * ch, ch)]], rows, sem)
            if c >= 1:
                prev_rows, _ = bufs[(c - 1) % 2]
                copies[c - 1].wait()
                pltpu.sync_copy(prev_rows,
                                out_hbm.at[pl.ds(base + (c - 1) * ch, ch)])
        copies[n_ch - 1].wait()
        last_rows, _ = bufs[(n_ch - 1) % 2]
        pltpu.sync_copy(last_rows,
                        out_hbm.at[pl.ds(base + (n_ch - 1) * ch, ch)])

    return gather_kernel(wt, idx)


def _transpose_body(q_ref, o_ref):
    o_ref[...] = jnp.transpose(q_ref[...], (0, 2, 1))


def _transpose_call(q3):
    b, p, d = q3.shape
    bs = 4
    return pl.pallas_call(
        _transpose_body,
        grid=(b // bs,),
        in_specs=[pl.BlockSpec((bs, p, d), lambda i: (i, 0, 0))],
        out_specs=pl.BlockSpec((bs, d, p), lambda i: (i, 0, 0)),
        out_shape=jax.ShapeDtypeStruct((b, d, p), jnp.float32),
    )(q3)


def kernel(x, weight):
    b, d, h, w = x.shape
    p = h * w
    x3 = x.reshape(b, d, p)
    idx3, wt = _argmin_call(x3, weight)
    quant = _sc_gather(wt, idx3.reshape(b * p))
    res3 = _transpose_call(quant.reshape(b, p, d))
    return res3.reshape(b, d, h, w), idx3.reshape(b, h, w)


# speedup vs baseline: 1.1745x; 1.0172x over previous
import functools

import jax
import jax.numpy as jnp
from jax import lax
from jax.experimental import pallas as pl
from jax.experimental.pallas import tpu as pltpu
from jax.experimental.pallas import tpu_sc as plsc


def _argmin_body(k_codes, x_ref, w_ref, idx_ref, wt_ref):
    xb = x_ref[0]
    w = w_ref[...]
    st = lax.dot_general(w, xb, (((0,), (0,)), ((), ())),
                         preferred_element_type=jnp.float32)
    x2 = jnp.sum(xb * xb, axis=0)
    e2 = jnp.sum(w * w, axis=0)
    dist = (x2[None, :] - 2.0 * st) + e2[:, None]
    m = jnp.min(dist, axis=0, keepdims=True)
    kiota = lax.broadcasted_iota(jnp.int32, dist.shape, 0)
    am = jnp.min(jnp.where(dist == m, kiota, k_codes), axis=0)
    idx_ref[0, 0, :] = am

    @pl.when(pl.program_id(0) == 0)
    def _():
        wt_ref[...] = w.T


def _argmin_call(x3, weight):
    b, d, p = x3.shape
    k = weight.shape[1]
    return pl.pallas_call(
        functools.partial(_argmin_body, k),
        grid=(b,),
        in_specs=[
            pl.BlockSpec((1, d, p), lambda i: (i, 0, 0)),
            pl.BlockSpec((d, k), lambda i: (0, 0)),
        ],
        out_specs=[
            pl.BlockSpec((1, 1, p), lambda i: (i, 0, 0)),
            pl.BlockSpec((k, d), lambda i: (0, 0)),
        ],
        out_shape=[
            jax.ShapeDtypeStruct((b, 1, p), jnp.int32),
            jax.ShapeDtypeStruct((k, d), jnp.float32),
        ],
    )(x3, weight)


def _sc_gather(wt, idx):
    nc, ns = 2, 16
    nw = nc * ns
    n, d = idx.shape[0], wt.shape[1]
    b_per_w = n // nw
    ch = min(128, b_per_w)
    n_ch = b_per_w // ch
    mesh = plsc.VectorSubcoreMesh(core_axis_name="c", subcore_axis_name="s",
                                  num_cores=nc, num_subcores=ns)

    nbuf = min(3, n_ch)

    @functools.partial(
        pl.kernel, mesh=mesh,
        out_type=jax.ShapeDtypeStruct((n, d), jnp.float32),
        scratch_types=(
            [pltpu.VMEM((b_per_w,), jnp.int32)]
            + [pltpu.VMEM((ch, d), jnp.float32)] * nbuf
            + [pltpu.SemaphoreType.DMA] * (2 * nbuf)
        ),
    )
    def gather_kernel(table_hbm, idx_hbm, out_hbm, idx_v, *bufs_sems):
        rows_bufs = bufs_sems[:nbuf]
        gsems = bufs_sems[nbuf:2 * nbuf]
        osems = bufs_sems[2 * nbuf:]
        wid = lax.axis_index("s") * nc + lax.axis_index("c")
        base = wid * b_per_w
        pltpu.sync_copy(idx_hbm.at[pl.ds(base, b_per_w)], idx_v)
        gathers = [None] * n_ch
        outs = [None] * n_ch
        for c in range(n_ch):
            if c >= nbuf:
                outs[c - nbuf].wait()
            gathers[c] = pltpu.async_copy(
                table_hbm.at[idx_v.at[pl.ds(c * ch, ch)]],
                rows_bufs[c % nbuf], gsems[c % nbuf])
            if c >= 1:
                gathers[c - 1].wait()
                outs[c - 1] = pltpu.async_copy(
                    rows_bufs[(c - 1) % nbuf],
                    out_hbm.at[pl.ds(base + (c - 1) * ch, ch)],
                    osems[(c - 1) % nbuf])
        gathers[n_ch - 1].wait()
        outs[n_ch - 1] = pltpu.async_copy(
            rows_bufs[(n_ch - 1) % nbuf],
            out_hbm.at[pl.ds(base + (n_ch - 1) * ch, ch)],
            osems[(n_ch - 1) % nbuf])
        for c in range(max(0, n_ch - nbuf), n_ch):
            outs[c].wait()

    return gather_kernel(wt, idx)


def _transpose_body(q_ref, o_ref):
    o_ref[...] = jnp.transpose(q_ref[...], (0, 2, 1))


def _transpose_call(q3):
    b, p, d = q3.shape
    bs = 8
    return pl.pallas_call(
        _transpose_body,
        grid=(b // bs,),
        in_specs=[pl.BlockSpec((bs, p, d), lambda i: (i, 0, 0))],
        out_specs=pl.BlockSpec((bs, d, p), lambda i: (i, 0, 0)),
        out_shape=jax.ShapeDtypeStruct((b, d, p), jnp.float32),
    )(q3)


def kernel(x, weight):
    b, d, h, w = x.shape
    p = h * w
    x3 = x.reshape(b, d, p)
    idx3, wt = _argmin_call(x3, weight)
    quant = _sc_gather(wt, idx3.reshape(b * p))
    res3 = _transpose_call(quant.reshape(b, p, d))
    return res3.reshape(b, d, h, w), idx3.reshape(b, h, w)


# speedup vs baseline: 1.1844x; 1.0084x over previous
import functools

import jax
import jax.numpy as jnp
from jax import lax
from jax.experimental import pallas as pl
from jax.experimental.pallas import tpu as pltpu
from jax.experimental.pallas import tpu_sc as plsc


def _argmin_body(k_codes, bb, x_ref, w_ref, idx_ref, wt_ref):
    w = w_ref[...]
    e2 = jnp.sum(w * w, axis=0)
    for j in range(bb):
        xb = x_ref[j]
        st = lax.dot_general(w, xb, (((0,), (0,)), ((), ())),
                             preferred_element_type=jnp.float32)
        x2 = jnp.sum(xb * xb, axis=0)
        dist = (x2[None, :] - 2.0 * st) + e2[:, None]
        m = jnp.min(dist, axis=0, keepdims=True)
        kiota = lax.broadcasted_iota(jnp.int32, dist.shape, 0)
        am = jnp.min(jnp.where(dist == m, kiota, k_codes), axis=0)
        idx_ref[j, 0, :] = am

    @pl.when(pl.program_id(0) == 0)
    def _():
        wt_ref[...] = w.T


def _argmin_call(x3, weight):
    b, d, p = x3.shape
    k = weight.shape[1]
    bb = 2
    return pl.pallas_call(
        functools.partial(_argmin_body, k, bb),
        grid=(b // bb,),
        in_specs=[
            pl.BlockSpec((bb, d, p), lambda i: (i, 0, 0)),
            pl.BlockSpec((d, k), lambda i: (0, 0)),
        ],
        out_specs=[
            pl.BlockSpec((bb, 1, p), lambda i: (i, 0, 0)),
            pl.BlockSpec((k, d), lambda i: (0, 0)),
        ],
        out_shape=[
            jax.ShapeDtypeStruct((b, 1, p), jnp.int32),
            jax.ShapeDtypeStruct((k, d), jnp.float32),
        ],
    )(x3, weight)


def _sc_gather(wt, idx):
    nc, ns = 2, 16
    nw = nc * ns
    n, d = idx.shape[0], wt.shape[1]
    b_per_w = n // nw
    ch = min(128, b_per_w)
    n_ch = b_per_w // ch
    mesh = plsc.VectorSubcoreMesh(core_axis_name="c", subcore_axis_name="s",
                                  num_cores=nc, num_subcores=ns)

    nbuf = min(3, n_ch)

    @functools.partial(
        pl.kernel, mesh=mesh,
        out_type=jax.ShapeDtypeStruct((n, d), jnp.float32),
        scratch_types=(
            [pltpu.VMEM((b_per_w,), jnp.int32)]
            + [pltpu.VMEM((ch, d), jnp.float32)] * nbuf
            + [pltpu.SemaphoreType.DMA] * (2 * nbuf)
        ),
    )
    def gather_kernel(table_hbm, idx_hbm, out_hbm, idx_v, *bufs_sems):
        rows_bufs = bufs_sems[:nbuf]
        gsems = bufs_sems[nbuf:2 * nbuf]
        osems = bufs_sems[2 * nbuf:]
        wid = lax.axis_index("s") * nc + lax.axis_index("c")
        base = wid * b_per_w
        pltpu.sync_copy(idx_hbm.at[pl.ds(base, b_per_w)], idx_v)
        gathers = [None] * n_ch
        outs = [None] * n_ch
        for c in range(n_ch):
            if c >= nbuf:
                outs[c - nbuf].wait()
            gathers[c] = pltpu.async_copy(
                table_hbm.at[idx_v.at[pl.ds(c * ch, ch)]],
                rows_bufs[c % nbuf], gsems[c % nbuf])
            if c >= 1:
                gathers[c - 1].wait()
                outs[c - 1] = pltpu.async_copy(
                    rows_bufs[(c - 1) % nbuf],
                    out_hbm.at[pl.ds(base + (c - 1) * ch, ch)],
                    osems[(c - 1) % nbuf])
        gathers[n_ch - 1].wait()
        outs[n_ch - 1] = pltpu.async_copy(
            rows_bufs[(n_ch - 1) % nbuf],
            out_hbm.at[pl.ds(base + (n_ch - 1) * ch, ch)],
            osems[(n_ch - 1) % nbuf])
        for c in range(max(0, n_ch - nbuf), n_ch):
            outs[c].wait()

    return gather_kernel(wt, idx)


def _transpose_body(q_ref, o_ref):
    o_ref[...] = jnp.transpose(q_ref[...], (0, 2, 1))


def _transpose_call(q3):
    b, p, d = q3.shape
    bs = 8
    return pl.pallas_call(
        _transpose_body,
        grid=(b // bs,),
        in_specs=[pl.BlockSpec((bs, p, d), lambda i: (i, 0, 0))],
        out_specs=pl.BlockSpec((bs, d, p), lambda i: (i, 0, 0)),
        out_shape=jax.ShapeDtypeStruct((b, d, p), jnp.float32),
    )(q3)


def kernel(x, weight):
    b, d, h, w = x.shape
    p = h * w
    x3 = x.reshape(b, d, p)
    idx3, wt = _argmin_call(x3, weight)
    quant = _sc_gather(wt, idx3.reshape(b * p))
    res3 = _transpose_call(quant.reshape(b, p, d))
    return res3.reshape(b, d, h, w), idx3.reshape(b, h, w)
